# baseline (speedup 1.0000x reference)
import functools

import jax
import jax.numpy as jnp
from jax import lax
from jax.experimental import pallas as pl
from jax.experimental.pallas import tpu as pltpu
from jax.experimental.pallas import tpu_sc as plsc

_N = 10000
_E = 160000
_D = 256
_EPS = 1e-5


_ROWS = 1000


def _ffn_ln_body(x_ref, w1t_ref, b1_ref, w2t_ref, b2_ref, g_ref, b_ref, o_ref):
    x = x_ref[...]
    h = jnp.maximum(
        jnp.dot(x, w1t_ref[...], preferred_element_type=jnp.float32) + b1_ref[...], 0.0
    )
    ff = jnp.dot(h, w2t_ref[...], preferred_element_type=jnp.float32) + b2_ref[...]
    y = ff + x
    m = jnp.mean(y, axis=-1, keepdims=True)
    v = jnp.mean((y - m) ** 2, axis=-1, keepdims=True)
    o_ref[...] = (y - m) * lax.rsqrt(v + _EPS) * g_ref[...] + b_ref[...]


def _gcn_apply_body(agg_ref, ff_ref, wt_ref, b_ref, g_ref, bb_ref, o_ref):
    attn = jnp.maximum(
        jnp.dot(agg_ref[...], wt_ref[...], preferred_element_type=jnp.float32)
        + b_ref[...],
        0.0,
    )
    y = attn + ff_ref[...]
    m = jnp.mean(y, axis=-1, keepdims=True)
    v = jnp.mean((y - m) ** 2, axis=-1, keepdims=True)
    o_ref[...] = (y - m) * lax.rsqrt(v + _EPS) * g_ref[...] + bb_ref[...]


def _row_spec():
    return pl.BlockSpec((_ROWS, _D), lambda i: (i, 0))


def _full_spec():
    return pl.BlockSpec((_D, _D), lambda i: (0, 0))


def _vec_spec():
    return pl.BlockSpec((1, _D), lambda i: (0, 0))


def _ffn_ln(x, w1t, b1, w2t, b2, g, b):
    return pl.pallas_call(
        _ffn_ln_body,
        grid=(_N // _ROWS,),
        in_specs=[
            _row_spec(),
            _full_spec(),
            _vec_spec(),
            _full_spec(),
            _vec_spec(),
            _vec_spec(),
            _vec_spec(),
        ],
        out_specs=_row_spec(),
        out_shape=jax.ShapeDtypeStruct((_N, _D), jnp.float32),
    )(x, w1t, b1, w2t, b2, g, b)


def _gcn_apply(agg, ff, wt, b, g, bb):
    return pl.pallas_call(
        _gcn_apply_body,
        grid=(_N // _ROWS,),
        in_specs=[
            _row_spec(),
            _row_spec(),
            _full_spec(),
            _vec_spec(),
            _vec_spec(),
            _vec_spec(),
        ],
        out_specs=_row_spec(),
        out_shape=jax.ShapeDtypeStruct((_N, _D), jnp.float32),
    )(agg, ff, wt, b, g, bb)


_NC = 2
_NS = 16
_NW = _NC * _NS
_OWN = 320
_AROWS = _NW * _OWN
_ACC_R = _OWN + 1
_LTRASH = _OWN
_W = 32
_ECHUNK = 4096
_NPAIR = 19
_EREST = _E - _NPAIR * 2 * _ECHUNK
_TH = 4096
_PEND = _TH + 2 * _ECHUNK + _W
_MAXW = (_PEND + _W - 1) // _W


def _mp_body(ff_hbm, src_hbm, dst_hbm, agg_hbm, cnt_hbm,
             acc, rows, sbufA, dbufA, sbufB, dbufB,
             pend, win_srcs, cstage, csmem, semgA, semgB, semcA, semcB):
    c = lax.axis_index("c")
    s = lax.axis_index("s")
    wid = c * _NS + s
    lo = wid * _OWN
    lanes = lax.broadcasted_iota(jnp.int32, (16,), 0)

    def zero(r, carry):
        for k in range(_D // 16):
            acc[r, pl.ds(k * 16, 16)] = jnp.zeros((16,), jnp.float32)
        return carry

    lax.fori_loop(0, _ACC_R, zero, 0)

    def fire_gather(w, slot):
        for k in range(_W // 16):
            packed = pend[pl.ds(w * _W + k * 16, 16)]
            win_srcs[slot, pl.ds(k * 16, 16)] = packed >> 9
        sem = semgA if slot == 0 else semgB
        pltpu.async_copy(ff_hbm.at[win_srcs.at[slot]],
                         rows.at[pl.ds(slot * _W, _W)], sem)

    def wait_gather(slot):
        sem = semgA if slot == 0 else semgB
        pltpu.make_async_copy(ff_hbm.at[win_srcs.at[slot]],
                              rows.at[pl.ds(slot * _W, _W)], sem).wait()

    def accumulate(w, slot):
        def grp(g, carry2):
            locv = pend[pl.ds(w * _W + g * 16, 16)] & 511
            rowv = g * 16 + lanes + slot * _W

            def col(e, carry3):
                ev = jnp.full((16,), 0, jnp.int32) + e
                vals = plsc.load_gather(rows, [rowv, ev])
                plsc.addupdate_scatter(acc, [locv, ev], vals)
                return carry3

            lax.fori_loop(0, _D, col, 0)
            return carry2

        lax.fori_loop(0, _W // 16, grp, 0)

    def drain(nw):
        @pl.when(nw >= 1)
        def _():
            fire_gather(0, 0)

        @pl.when(nw >= 2)
        def _():
            fire_gather(1, 1)

        def pairw(j, carry):
            w0 = 2 * j

            @pl.when(w0 < nw)
            def _():
                wait_gather(0)
                accumulate(w0, 0)

                @pl.when(w0 + 2 < nw)
                def _():
                    fire_gather(w0 + 2, 0)

            w1 = w0 + 1

            @pl.when(w1 < nw)
            def _():
                wait_gather(1)
                accumulate(w1, 1)

                @pl.when(w1 + 2 < nw)
                def _():
                    fire_gather(w1 + 2, 1)

            return carry

        lax.fori_loop(0, (_MAXW + 1) // 2, pairw, 0)

    def scan_buf(n16, sb, db, cntv):
        def body(i, cntv):
            dv = db[pl.ds(i * 16, 16)]
            sv = sb[pl.ds(i * 16, 16)]
            m = (dv >= lo) & (dv < lo + _OWN)
            mi = jnp.where(m, 1, 0)
            packed = sv * 512 + jnp.where(m, dv - lo, _LTRASH)
            pc = mi
            for d in (1, 2, 4, 8):
                sh = jnp.take(pc, jnp.maximum(lanes - d, 0))
                pc = pc + jnp.where(lanes >= d, sh, 0)
            pos = jnp.where(m, cntv + pc - 1, _PEND)
            plsc.store_scatter(pend, [pos], packed)
            tot = jnp.take(pc, jnp.full((16,), 15, jnp.int32))
            return cntv + tot

        return lax.fori_loop(0, n16, body, cntv)

    def fire_chunk(eb, sb, db, sem):
        pltpu.async_copy(src_hbm.at[pl.ds(eb, _ECHUNK)], sb, sem)
        pltpu.async_copy(dst_hbm.at[pl.ds(eb, _ECHUNK)], db, sem)

    def wait_chunk(eb, sb, db, sem):
        pltpu.make_async_copy(src_hbm.at[pl.ds(eb, _ECHUNK)], sb, sem).wait()
        pltpu.make_async_copy(dst_hbm.at[pl.ds(eb, _ECHUNK)], db, sem).wait()

    def bounce_cnt(cntv):
        return cntv[0]

    def drain_full(cntv):
        cnt_s = bounce_cnt(cntv)
        nw = jnp.where(cnt_s >= _TH, lax.div(cnt_s, _W), 0)
        drain(nw)
        base = nw * _W
        for k in range(_W // 16):
            pend[pl.ds(k * 16, 16)] = pend[pl.ds(base + k * 16, 16)]
        return cntv - base

    fire_chunk(0, sbufA, dbufA, semcA)
    zero16 = jnp.zeros((16,), jnp.int32)

    def chunk_pair(p, cntv):
        eb = p * 2 * _ECHUNK
        fire_chunk(eb + _ECHUNK, sbufB, dbufB, semcB)
        wait_chunk(eb, sbufA, dbufA, semcA)
        cntv = scan_buf(_ECHUNK // 16, sbufA, dbufA, cntv)

        @pl.when(p + 1 < _NPAIR)
        def _():
            fire_chunk(eb + 2 * _ECHUNK, sbufA, dbufA, semcA)

        wait_chunk(eb + _ECHUNK, sbufB, dbufB, semcB)
        cntv = scan_buf(_ECHUNK // 16, sbufB, dbufB, cntv)
        return drain_full(cntv)

    cntv = lax.fori_loop(0, _NPAIR, chunk_pair, zero16)

    eb = _NPAIR * 2 * _ECHUNK
    pltpu.sync_copy(src_hbm.at[pl.ds(eb, _ECHUNK)], sbufA)
    pltpu.sync_copy(dst_hbm.at[pl.ds(eb, _ECHUNK)], dbufA)
    cntv = scan_buf(_ECHUNK // 16, sbufA, dbufA, cntv)
    eb2 = eb + _ECHUNK
    rest = _EREST - _ECHUNK
    pltpu.sync_copy(src_hbm.at[pl.ds(eb2, rest)], sbufB.at[pl.ds(0, rest)])
    pltpu.sync_copy(dst_hbm.at[pl.ds(eb2, rest)], dbufB.at[pl.ds(0, rest)])
    cntv = scan_buf(rest // 16, sbufB, dbufB, cntv)

    def trashpad(k, carry):
        posv = k * 16 + lanes
        v = pend[pl.ds(k * 16, 16)]
        pend[pl.ds(k * 16, 16)] = jnp.where(posv < cntv, v, _LTRASH)
        return carry

    lax.fori_loop(0, _PEND // 16, trashpad, 0)
    cnt_s = bounce_cnt(cntv)
    drain(lax.div(cnt_s + _W - 1, _W))

    pltpu.sync_copy(acc.at[pl.ds(0, _OWN)], agg_hbm.at[pl.ds(lo, _OWN)])


@functools.lru_cache(maxsize=1)
def _message_pass_kernel():
    return pl.kernel(
        _mp_body,
        out_type=(jax.ShapeDtypeStruct((_AROWS, _D), jnp.float32),
                  jax.ShapeDtypeStruct((_NW, 16), jnp.int32)),
        mesh=plsc.VectorSubcoreMesh(core_axis_name="c", subcore_axis_name="s",
                                    num_cores=_NC, num_subcores=_NS),
        compiler_params=pltpu.CompilerParams(needs_layout_passes=False),
        scratch_types=[
            pltpu.VMEM((_ACC_R, _D), jnp.float32),
            pltpu.VMEM((2 * _W, _D), jnp.float32),
            pltpu.VMEM((_ECHUNK,), jnp.int32),
            pltpu.VMEM((_ECHUNK,), jnp.int32),
            pltpu.VMEM((_ECHUNK,), jnp.int32),
            pltpu.VMEM((_ECHUNK,), jnp.int32),
            pltpu.VMEM((_PEND + 16,), jnp.int32),
            pltpu.VMEM((2, _W), jnp.int32),
            pltpu.VMEM((16,), jnp.int32),
            pltpu.SMEM((16,), jnp.int32),
            pltpu.SemaphoreType.DMA,
            pltpu.SemaphoreType.DMA,
            pltpu.SemaphoreType.DMA,
            pltpu.SemaphoreType.DMA,
        ],
    )


def _message_pass(ff, src, dst):
    agg, _ = _message_pass_kernel()(ff, src, dst)
    return agg[:_N]


def kernel(features, edge_index,
           l0_ff_w1, l0_ff_b1, l0_ff_w2, l0_ff_b2, l0_ffln_g, l0_ffln_b,
           l0_gcn_w, l0_gcn_b, l0_ln_g, l0_ln_b,
           l1_ff_w1, l1_ff_b1, l1_ff_w2, l1_ff_b2, l1_ffln_g, l1_ffln_b,
           l1_gcn_w, l1_gcn_b, l1_ln_g, l1_ln_b):
    src = edge_index[0]
    dst = edge_index[1]

    params = [
        (l0_ff_w1, l0_ff_b1, l0_ff_w2, l0_ff_b2, l0_ffln_g, l0_ffln_b,
         l0_gcn_w, l0_gcn_b, l0_ln_g, l0_ln_b),
        (l1_ff_w1, l1_ff_b1, l1_ff_w2, l1_ff_b2, l1_ffln_g, l1_ffln_b,
         l1_gcn_w, l1_gcn_b, l1_ln_g, l1_ln_b),
    ]

    out = features
    for (w1, b1, w2, b2, fg, fb, gw, gb, lg, lb) in params:
        ff = _ffn_ln(out, w1.T, b1.reshape(1, _D), w2.T, b2.reshape(1, _D),
                     fg.reshape(1, _D), fb.reshape(1, _D))
        agg = _message_pass(ff, src, dst)
        out = _gcn_apply(agg, ff, gw.T, gb.reshape(1, _D),
                         lg.reshape(1, _D), lb.reshape(1, _D))
    return out


# speedup vs baseline: 1.0218x; 1.0218x over previous
import functools

import jax
import jax.numpy as jnp
from jax import lax
from jax.experimental import pallas as pl
from jax.experimental.pallas import tpu as pltpu
from jax.experimental.pallas import tpu_sc as plsc

_N = 10000
_E = 160000
_D = 256
_EPS = 1e-5


_ROWS = 1000


def _ffn_ln_body(x_ref, w1t_ref, b1_ref, w2t_ref, b2_ref, g_ref, b_ref, o_ref):
    x = x_ref[...]
    h = jnp.maximum(
        jnp.dot(x, w1t_ref[...], preferred_element_type=jnp.float32) + b1_ref[...], 0.0
    )
    ff = jnp.dot(h, w2t_ref[...], preferred_element_type=jnp.float32) + b2_ref[...]
    y = ff + x
    m = jnp.mean(y, axis=-1, keepdims=True)
    v = jnp.mean((y - m) ** 2, axis=-1, keepdims=True)
    o_ref[...] = (y - m) * lax.rsqrt(v + _EPS) * g_ref[...] + b_ref[...]


def _gcn_apply_body(agg_ref, ff_ref, wt_ref, b_ref, g_ref, bb_ref, o_ref):
    attn = jnp.maximum(
        jnp.dot(agg_ref[...], wt_ref[...], preferred_element_type=jnp.float32)
        + b_ref[...],
        0.0,
    )
    y = attn + ff_ref[...]
    m = jnp.mean(y, axis=-1, keepdims=True)
    v = jnp.mean((y - m) ** 2, axis=-1, keepdims=True)
    o_ref[...] = (y - m) * lax.rsqrt(v + _EPS) * g_ref[...] + bb_ref[...]


def _row_spec():
    return pl.BlockSpec((_ROWS, _D), lambda i: (i, 0))


def _full_spec():
    return pl.BlockSpec((_D, _D), lambda i: (0, 0))


def _vec_spec():
    return pl.BlockSpec((1, _D), lambda i: (0, 0))


def _ffn_ln(x, w1t, b1, w2t, b2, g, b):
    return pl.pallas_call(
        _ffn_ln_body,
        grid=(_N // _ROWS,),
        in_specs=[
            _row_spec(),
            _full_spec(),
            _vec_spec(),
            _full_spec(),
            _vec_spec(),
            _vec_spec(),
            _vec_spec(),
        ],
        out_specs=_row_spec(),
        out_shape=jax.ShapeDtypeStruct((_N, _D), jnp.float32),
    )(x, w1t, b1, w2t, b2, g, b)


def _gcn_apply(agg, ff, wt, b, g, bb):
    return pl.pallas_call(
        _gcn_apply_body,
        grid=(_N // _ROWS,),
        in_specs=[
            _row_spec(),
            _row_spec(),
            _full_spec(),
            _vec_spec(),
            _vec_spec(),
            _vec_spec(),
        ],
        out_specs=_row_spec(),
        out_shape=jax.ShapeDtypeStruct((_N, _D), jnp.float32),
    )(agg, ff, wt, b, g, bb)


_NC = 2
_NS = 16
_NW = _NC * _NS
_OWN = 320
_AROWS = _NW * _OWN
_ACC_R = _OWN + 1
_LTRASH = _OWN
_W = 32
_ECHUNK = 4096
_NPAIR = 19
_EREST = _E - _NPAIR * 2 * _ECHUNK
_TH = 4096
_PEND = _TH + 2 * _ECHUNK + _W
_MAXW = (_PEND + _W - 1) // _W


def _mp_body(ff_hbm, src_hbm, dst_hbm, agg_hbm, cnt_hbm,
             acc, rows, sbufA, dbufA, sbufB, dbufB,
             pend, win_srcs, cstage, csmem, semgA, semgB, semcA, semcB):
    c = lax.axis_index("c")
    s = lax.axis_index("s")
    wid = c * _NS + s
    lo = wid * _OWN
    lanes = lax.broadcasted_iota(jnp.int32, (16,), 0)

    def zero(r, carry):
        for k in range(_D // 16):
            acc[r, pl.ds(k * 16, 16)] = jnp.zeros((16,), jnp.float32)
        return carry

    lax.fori_loop(0, _ACC_R, zero, 0)

    def fire_gather(w, slot):
        for k in range(_W // 16):
            packed = pend[pl.ds(w * _W + k * 16, 16)]
            win_srcs[slot, pl.ds(k * 16, 16)] = packed >> 9
        sem = semgA if slot == 0 else semgB
        pltpu.async_copy(ff_hbm.at[win_srcs.at[slot]],
                         rows.at[pl.ds(slot * _W, _W)], sem)

    def wait_gather(slot):
        sem = semgA if slot == 0 else semgB
        pltpu.make_async_copy(ff_hbm.at[win_srcs.at[slot]],
                              rows.at[pl.ds(slot * _W, _W)], sem).wait()

    def accumulate(w, slot):
        def grp(g, carry2):
            locv = pend[pl.ds(w * _W + g * 16, 16)] & 511
            rowv = g * 16 + lanes + slot * _W

            def col(e8, carry3):
                for u in range(8):
                    ev = jnp.full((16,), 0, jnp.int32) + (e8 * 8 + u)
                    vals = plsc.load_gather(rows, [rowv, ev])
                    plsc.addupdate_scatter(acc, [locv, ev], vals)
                return carry3

            lax.fori_loop(0, _D // 8, col, 0)
            return carry2

        lax.fori_loop(0, _W // 16, grp, 0)

    def drain(nw):
        @pl.when(nw >= 1)
        def _():
            fire_gather(0, 0)

        @pl.when(nw >= 2)
        def _():
            fire_gather(1, 1)

        def pairw(j, carry):
            w0 = 2 * j

            @pl.when(w0 < nw)
            def _():
                wait_gather(0)
                accumulate(w0, 0)

                @pl.when(w0 + 2 < nw)
                def _():
                    fire_gather(w0 + 2, 0)

            w1 = w0 + 1

            @pl.when(w1 < nw)
            def _():
                wait_gather(1)
                accumulate(w1, 1)

                @pl.when(w1 + 2 < nw)
                def _():
                    fire_gather(w1 + 2, 1)

            return carry

        lax.fori_loop(0, lax.div(nw + 1, 2), pairw, 0)

    def scan_buf(n16, sb, db, cntv):
        def body(i, cntv):
            dv = db[pl.ds(i * 16, 16)]
            sv = sb[pl.ds(i * 16, 16)]
            m = (dv >= lo) & (dv < lo + _OWN)
            mi = jnp.where(m, 1, 0)
            packed = sv * 512 + jnp.where(m, dv - lo, _LTRASH)
            pc = mi
            for d in (1, 2, 4, 8):
                sh = jnp.take(pc, jnp.maximum(lanes - d, 0))
                pc = pc + jnp.where(lanes >= d, sh, 0)
            pos = jnp.where(m, cntv + pc - 1, _PEND)
            plsc.store_scatter(pend, [pos], packed)
            tot = jnp.take(pc, jnp.full((16,), 15, jnp.int32))
            return cntv + tot

        return lax.fori_loop(0, n16, body, cntv)

    def fire_chunk(eb, sb, db, sem):
        pltpu.async_copy(src_hbm.at[pl.ds(eb, _ECHUNK)], sb, sem)
        pltpu.async_copy(dst_hbm.at[pl.ds(eb, _ECHUNK)], db, sem)

    def wait_chunk(eb, sb, db, sem):
        pltpu.make_async_copy(src_hbm.at[pl.ds(eb, _ECHUNK)], sb, sem).wait()
        pltpu.make_async_copy(dst_hbm.at[pl.ds(eb, _ECHUNK)], db, sem).wait()

    def bounce_cnt(cntv):
        return cntv[0]

    def drain_full(cntv):
        cnt_s = bounce_cnt(cntv)
        nw = jnp.where(cnt_s >= _TH, lax.div(cnt_s, _W), 0)
        drain(nw)
        base = nw * _W
        for k in range(_W // 16):
            pend[pl.ds(k * 16, 16)] = pend[pl.ds(base + k * 16, 16)]
        return cntv - base

    fire_chunk(0, sbufA, dbufA, semcA)
    zero16 = jnp.zeros((16,), jnp.int32)

    def chunk_pair(p, cntv):
        eb = p * 2 * _ECHUNK
        fire_chunk(eb + _ECHUNK, sbufB, dbufB, semcB)
        wait_chunk(eb, sbufA, dbufA, semcA)
        cntv = scan_buf(_ECHUNK // 16, sbufA, dbufA, cntv)

        @pl.when(p + 1 < _NPAIR)
        def _():
            fire_chunk(eb + 2 * _ECHUNK, sbufA, dbufA, semcA)

        wait_chunk(eb + _ECHUNK, sbufB, dbufB, semcB)
        cntv = scan_buf(_ECHUNK // 16, sbufB, dbufB, cntv)
        return drain_full(cntv)

    cntv = lax.fori_loop(0, _NPAIR, chunk_pair, zero16)

    eb = _NPAIR * 2 * _ECHUNK
    pltpu.sync_copy(src_hbm.at[pl.ds(eb, _ECHUNK)], sbufA)
    pltpu.sync_copy(dst_hbm.at[pl.ds(eb, _ECHUNK)], dbufA)
    cntv = scan_buf(_ECHUNK // 16, sbufA, dbufA, cntv)
    eb2 = eb + _ECHUNK
    rest = _EREST - _ECHUNK
    pltpu.sync_copy(src_hbm.at[pl.ds(eb2, rest)], sbufB.at[pl.ds(0, rest)])
    pltpu.sync_copy(dst_hbm.at[pl.ds(eb2, rest)], dbufB.at[pl.ds(0, rest)])
    cntv = scan_buf(rest // 16, sbufB, dbufB, cntv)

    def trashpad(k, carry):
        posv = k * 16 + lanes
        v = pend[pl.ds(k * 16, 16)]
        pend[pl.ds(k * 16, 16)] = jnp.where(posv < cntv, v, _LTRASH)
        return carry

    lax.fori_loop(0, _PEND // 16, trashpad, 0)
    cnt_s = bounce_cnt(cntv)
    drain(lax.div(cnt_s + _W - 1, _W))

    pltpu.sync_copy(acc.at[pl.ds(0, _OWN)], agg_hbm.at[pl.ds(lo, _OWN)])


@functools.lru_cache(maxsize=1)
def _message_pass_kernel():
    return pl.kernel(
        _mp_body,
        out_type=(jax.ShapeDtypeStruct((_AROWS, _D), jnp.float32),
                  jax.ShapeDtypeStruct((_NW, 16), jnp.int32)),
        mesh=plsc.VectorSubcoreMesh(core_axis_name="c", subcore_axis_name="s",
                                    num_cores=_NC, num_subcores=_NS),
        compiler_params=pltpu.CompilerParams(needs_layout_passes=False),
        scratch_types=[
            pltpu.VMEM((_ACC_R, _D), jnp.float32),
            pltpu.VMEM((2 * _W, _D), jnp.float32),
            pltpu.VMEM((_ECHUNK,), jnp.int32),
            pltpu.VMEM((_ECHUNK,), jnp.int32),
            pltpu.VMEM((_ECHUNK,), jnp.int32),
            pltpu.VMEM((_ECHUNK,), jnp.int32),
            pltpu.VMEM((_PEND + 16,), jnp.int32),
            pltpu.VMEM((2, _W), jnp.int32),
            pltpu.VMEM((16,), jnp.int32),
            pltpu.SMEM((16,), jnp.int32),
            pltpu.SemaphoreType.DMA,
            pltpu.SemaphoreType.DMA,
            pltpu.SemaphoreType.DMA,
            pltpu.SemaphoreType.DMA,
        ],
    )


def _message_pass(ff, src, dst):
    agg, _ = _message_pass_kernel()(ff, src, dst)
    return agg[:_N]


def kernel(features, edge_index,
           l0_ff_w1, l0_ff_b1, l0_ff_w2, l0_ff_b2, l0_ffln_g, l0_ffln_b,
           l0_gcn_w, l0_gcn_b, l0_ln_g, l0_ln_b,
           l1_ff_w1, l1_ff_b1, l1_ff_w2, l1_ff_b2, l1_ffln_g, l1_ffln_b,
           l1_gcn_w, l1_gcn_b, l1_ln_g, l1_ln_b):
    src = edge_index[0]
    dst = edge_index[1]

    params = [
        (l0_ff_w1, l0_ff_b1, l0_ff_w2, l0_ff_b2, l0_ffln_g, l0_ffln_b,
         l0_gcn_w, l0_gcn_b, l0_ln_g, l0_ln_b),
        (l1_ff_w1, l1_ff_b1, l1_ff_w2, l1_ff_b2, l1_ffln_g, l1_ffln_b,
         l1_gcn_w, l1_gcn_b, l1_ln_g, l1_ln_b),
    ]

    out = features
    for (w1, b1, w2, b2, fg, fb, gw, gb, lg, lb) in params:
        ff = _ffn_ln(out, w1.T, b1.reshape(1, _D), w2.T, b2.reshape(1, _D),
                     fg.reshape(1, _D), fb.reshape(1, _D))
        agg = _message_pass(ff, src, dst)
        out = _gcn_apply(agg, ff, gw.T, gb.reshape(1, _D),
                         lg.reshape(1, _D), lb.reshape(1, _D))
    return out


# speedup vs baseline: 1.0846x; 1.0615x over previous
import functools

import jax
import jax.numpy as jnp
from jax import lax
from jax.experimental import pallas as pl
from jax.experimental.pallas import tpu as pltpu
from jax.experimental.pallas import tpu_sc as plsc

_N = 10000
_E = 160000
_D = 256
_EPS = 1e-5


_ROWS = 1000


def _ffn_ln_body(x_ref, w1t_ref, b1_ref, w2t_ref, b2_ref, g_ref, b_ref, o_ref):
    x = x_ref[...]
    h = jnp.maximum(
        jnp.dot(x, w1t_ref[...], preferred_element_type=jnp.float32) + b1_ref[...], 0.0
    )
    ff = jnp.dot(h, w2t_ref[...], preferred_element_type=jnp.float32) + b2_ref[...]
    y = ff + x
    m = jnp.mean(y, axis=-1, keepdims=True)
    v = jnp.mean((y - m) ** 2, axis=-1, keepdims=True)
    o_ref[...] = (y - m) * lax.rsqrt(v + _EPS) * g_ref[...] + b_ref[...]


def _gcn_apply_body(agg_ref, ff_ref, wt_ref, b_ref, g_ref, bb_ref, o_ref):
    attn = jnp.maximum(
        jnp.dot(agg_ref[...], wt_ref[...], preferred_element_type=jnp.float32)
        + b_ref[...],
        0.0,
    )
    y = attn + ff_ref[...]
    m = jnp.mean(y, axis=-1, keepdims=True)
    v = jnp.mean((y - m) ** 2, axis=-1, keepdims=True)
    o_ref[...] = (y - m) * lax.rsqrt(v + _EPS) * g_ref[...] + bb_ref[...]


def _row_spec():
    return pl.BlockSpec((_ROWS, _D), lambda i: (i, 0))


def _full_spec():
    return pl.BlockSpec((_D, _D), lambda i: (0, 0))


def _vec_spec():
    return pl.BlockSpec((1, _D), lambda i: (0, 0))


def _ffn_ln(x, w1t, b1, w2t, b2, g, b):
    return pl.pallas_call(
        _ffn_ln_body,
        grid=(_N // _ROWS,),
        in_specs=[
            _row_spec(),
            _full_spec(),
            _vec_spec(),
            _full_spec(),
            _vec_spec(),
            _vec_spec(),
            _vec_spec(),
        ],
        out_specs=_row_spec(),
        out_shape=jax.ShapeDtypeStruct((_N, _D), jnp.float32),
    )(x, w1t, b1, w2t, b2, g, b)


def _gcn_apply(agg, ff, wt, b, g, bb):
    return pl.pallas_call(
        _gcn_apply_body,
        grid=(_N // _ROWS,),
        in_specs=[
            _row_spec(),
            _row_spec(),
            _full_spec(),
            _vec_spec(),
            _vec_spec(),
            _vec_spec(),
        ],
        out_specs=_row_spec(),
        out_shape=jax.ShapeDtypeStruct((_N, _D), jnp.float32),
    )(agg, ff, wt, b, g, bb)


_NC = 2
_NS = 16
_NW = _NC * _NS
_OWN = 320
_AROWS = _NW * _OWN
_ACC_R = _OWN + 1
_LTRASH = _OWN
_W = 48
_ECHUNK = 2048
_NPAIR = 39
_EREST = _E - _NPAIR * 2 * _ECHUNK
_TH = 2048
_PEND = _TH + 2 * _ECHUNK + _W
_MAXW = (_PEND + _W - 1) // _W


def _mp_body(ff_hbm, src_hbm, dst_hbm, agg_hbm, cnt_hbm,
             acc, rows, sbufA, dbufA, sbufB, dbufB,
             pend, win_srcs, cstage, csmem, semgA, semgB, semcA, semcB):
    c = lax.axis_index("c")
    s = lax.axis_index("s")
    wid = c * _NS + s
    lo = wid * _OWN
    lanes = lax.broadcasted_iota(jnp.int32, (16,), 0)

    def zero(r, carry):
        for k in range(_D // 16):
            acc[r, pl.ds(k * 16, 16)] = jnp.zeros((16,), jnp.float32)
        return carry

    lax.fori_loop(0, _ACC_R, zero, 0)

    def fire_gather(w, slot):
        for k in range(_W // 16):
            packed = pend[pl.ds(w * _W + k * 16, 16)]
            win_srcs[slot, pl.ds(k * 16, 16)] = packed >> 9
        sem = semgA if slot == 0 else semgB
        pltpu.async_copy(ff_hbm.at[win_srcs.at[slot]],
                         rows.at[pl.ds(slot * _W, _W)], sem)

    def wait_gather(slot):
        sem = semgA if slot == 0 else semgB
        pltpu.make_async_copy(ff_hbm.at[win_srcs.at[slot]],
                              rows.at[pl.ds(slot * _W, _W)], sem).wait()

    def accumulate(w, slot):
        def grp(g, carry2):
            locv = pend[pl.ds(w * _W + g * 16, 16)] & 511
            rowv = g * 16 + lanes + slot * _W

            def col(e8, carry3):
                for u in range(8):
                    ev = jnp.full((16,), 0, jnp.int32) + (e8 * 8 + u)
                    vals = plsc.load_gather(rows, [rowv, ev])
                    plsc.addupdate_scatter(acc, [locv, ev], vals)
                return carry3

            lax.fori_loop(0, _D // 8, col, 0)
            return carry2

        lax.fori_loop(0, _W // 16, grp, 0)

    def drain(nw):
        @pl.when(nw >= 1)
        def _():
            fire_gather(0, 0)

        @pl.when(nw >= 2)
        def _():
            fire_gather(1, 1)

        def pairw(j, carry):
            w0 = 2 * j

            @pl.when(w0 < nw)
            def _():
                wait_gather(0)
                accumulate(w0, 0)

                @pl.when(w0 + 2 < nw)
                def _():
                    fire_gather(w0 + 2, 0)

            w1 = w0 + 1

            @pl.when(w1 < nw)
            def _():
                wait_gather(1)
                accumulate(w1, 1)

                @pl.when(w1 + 2 < nw)
                def _():
                    fire_gather(w1 + 2, 1)

            return carry

        lax.fori_loop(0, lax.div(nw + 1, 2), pairw, 0)

    def scan_buf(n16, sb, db, cntv):
        lane15 = jnp.full((16,), 15, jnp.int32)

        def body(i, cntv):
            ms, packs, pcs = [], [], []
            for u in range(4):
                dv = db[pl.ds((i * 4 + u) * 16, 16)]
                sv = sb[pl.ds((i * 4 + u) * 16, 16)]
                m = (dv >= lo) & (dv < lo + _OWN)
                ms.append(m)
                packs.append(sv * 512 + jnp.where(m, dv - lo, _LTRASH))
                pcs.append(jnp.where(m, 1, 0))
            for d in (1, 2, 4, 8):
                for u in range(4):
                    sh = jnp.take(pcs[u], jnp.maximum(lanes - d, 0))
                    pcs[u] = pcs[u] + jnp.where(lanes >= d, sh, 0)
            base = cntv
            for u in range(4):
                pos = jnp.where(ms[u], base + pcs[u] - 1, _PEND)
                plsc.store_scatter(pend, [pos], packs[u])
                base = base + jnp.take(pcs[u], lane15)
            return base

        return lax.fori_loop(0, n16 // 4, body, cntv)

    def fire_chunk(eb, sb, db, sem):
        pltpu.async_copy(src_hbm.at[pl.ds(eb, _ECHUNK)], sb, sem)
        pltpu.async_copy(dst_hbm.at[pl.ds(eb, _ECHUNK)], db, sem)

    def wait_chunk(eb, sb, db, sem):
        pltpu.make_async_copy(src_hbm.at[pl.ds(eb, _ECHUNK)], sb, sem).wait()
        pltpu.make_async_copy(dst_hbm.at[pl.ds(eb, _ECHUNK)], db, sem).wait()

    def bounce_cnt(cntv):
        return cntv[0]

    def drain_full(cntv):
        cnt_s = bounce_cnt(cntv)
        nw = jnp.where(cnt_s >= _TH, lax.div(cnt_s, _W), 0)
        drain(nw)
        base = nw * _W
        for k in range(_W // 16):
            pend[pl.ds(k * 16, 16)] = pend[pl.ds(base + k * 16, 16)]
        return cntv - base

    fire_chunk(0, sbufA, dbufA, semcA)
    zero16 = jnp.zeros((16,), jnp.int32)

    def chunk_pair(p, cntv):
        eb = p * 2 * _ECHUNK
        fire_chunk(eb + _ECHUNK, sbufB, dbufB, semcB)
        wait_chunk(eb, sbufA, dbufA, semcA)
        cntv = scan_buf(_ECHUNK // 16, sbufA, dbufA, cntv)

        @pl.when(p + 1 < _NPAIR)
        def _():
            fire_chunk(eb + 2 * _ECHUNK, sbufA, dbufA, semcA)

        wait_chunk(eb + _ECHUNK, sbufB, dbufB, semcB)
        cntv = scan_buf(_ECHUNK // 16, sbufB, dbufB, cntv)
        return drain_full(cntv)

    cntv = lax.fori_loop(0, _NPAIR, chunk_pair, zero16)

    eb = _NPAIR * 2 * _ECHUNK
    pltpu.sync_copy(src_hbm.at[pl.ds(eb, _EREST)], sbufA.at[pl.ds(0, _EREST)])
    pltpu.sync_copy(dst_hbm.at[pl.ds(eb, _EREST)], dbufA.at[pl.ds(0, _EREST)])
    cntv = scan_buf(_EREST // 16, sbufA, dbufA, cntv)

    def trashpad(k, carry):
        posv = k * 16 + lanes
        v = pend[pl.ds(k * 16, 16)]
        pend[pl.ds(k * 16, 16)] = jnp.where(posv < cntv, v, _LTRASH)
        return carry

    lax.fori_loop(0, _PEND // 16, trashpad, 0)
    cnt_s = bounce_cnt(cntv)
    drain(lax.div(cnt_s + _W - 1, _W))

    pltpu.sync_copy(acc.at[pl.ds(0, _OWN)], agg_hbm.at[pl.ds(lo, _OWN)])


@functools.lru_cache(maxsize=1)
def _message_pass_kernel():
    return pl.kernel(
        _mp_body,
        out_type=(jax.ShapeDtypeStruct((_AROWS, _D), jnp.float32),
                  jax.ShapeDtypeStruct((_NW, 16), jnp.int32)),
        mesh=plsc.VectorSubcoreMesh(core_axis_name="c", subcore_axis_name="s",
                                    num_cores=_NC, num_subcores=_NS),
        compiler_params=pltpu.CompilerParams(needs_layout_passes=False),
        scratch_types=[
            pltpu.VMEM((_ACC_R, _D), jnp.float32),
            pltpu.VMEM((2 * _W, _D), jnp.float32),
            pltpu.VMEM((_ECHUNK,), jnp.int32),
            pltpu.VMEM((_ECHUNK,), jnp.int32),
            pltpu.VMEM((_ECHUNK,), jnp.int32),
            pltpu.VMEM((_ECHUNK,), jnp.int32),
            pltpu.VMEM((_PEND + 16,), jnp.int32),
            pltpu.VMEM((2, _W), jnp.int32),
            pltpu.VMEM((16,), jnp.int32),
            pltpu.SMEM((16,), jnp.int32),
            pltpu.SemaphoreType.DMA,
            pltpu.SemaphoreType.DMA,
            pltpu.SemaphoreType.DMA,
            pltpu.SemaphoreType.DMA,
        ],
    )


def _message_pass(ff, src, dst):
    agg, _ = _message_pass_kernel()(ff, src, dst)
    return agg[:_N]


def kernel(features, edge_index,
           l0_ff_w1, l0_ff_b1, l0_ff_w2, l0_ff_b2, l0_ffln_g, l0_ffln_b,
           l0_gcn_w, l0_gcn_b, l0_ln_g, l0_ln_b,
           l1_ff_w1, l1_ff_b1, l1_ff_w2, l1_ff_b2, l1_ffln_g, l1_ffln_b,
           l1_gcn_w, l1_gcn_b, l1_ln_g, l1_ln_b):
    src = edge_index[0]
    dst = edge_index[1]

    params = [
        (l0_ff_w1, l0_ff_b1, l0_ff_w2, l0_ff_b2, l0_ffln_g, l0_ffln_b,
         l0_gcn_w, l0_gcn_b, l0_ln_g, l0_ln_b),
        (l1_ff_w1, l1_ff_b1, l1_ff_w2, l1_ff_b2, l1_ffln_g, l1_ffln_b,
         l1_gcn_w, l1_gcn_b, l1_ln_g, l1_ln_b),
    ]

    out = features
    for (w1, b1, w2, b2, fg, fb, gw, gb, lg, lb) in params:
        ff = _ffn_ln(out, w1.T, b1.reshape(1, _D), w2.T, b2.reshape(1, _D),
                     fg.reshape(1, _D), fb.reshape(1, _D))
        agg = _message_pass(ff, src, dst)
        out = _gcn_apply(agg, ff, gw.T, gb.reshape(1, _D),
                         lg.reshape(1, _D), lb.reshape(1, _D))
    return out
